# baseline (speedup 1.0000x reference)
import functools

import jax
import jax.numpy as jnp
from jax.experimental import pallas as pl


def _bn_body(emb_ref, g_ref, b_ref, x32_ref, x16_ref):
    emb = emb_ref[...]
    mu = jnp.mean(emb, axis=0, keepdims=True)
    var = jnp.mean((emb - mu) ** 2, axis=0, keepdims=True)
    x = (emb - mu) * jax.lax.rsqrt(var + 1e-5) * g_ref[...] + b_ref[...]
    x32_ref[...] = x
    x16_ref[...] = x.astype(jnp.bfloat16)


def _mask_rows(arr, start, v):
    rows = jax.lax.broadcasted_iota(jnp.int32, arr.shape, 0) + start
    return jnp.where(rows < v, arr, 0.0)


def _passA_body(h_ref, x16_ref, eraw_ref, dege_ref, *, v, tv, nv):
    i = pl.program_id(0)
    h = h_ref[...]
    x16 = x16_ref[...]
    if v % tv:
        h, x16 = jax.lax.cond(
            i == nv - 1,
            lambda hh, xx: (_mask_rows(hh, i * tv, v), _mask_rows(xx, i * tv, v)),
            lambda hh, xx: (hh, xx),
            h, x16)
    de = jnp.sum(h, axis=0, keepdims=True)
    hb = h.astype(jnp.bfloat16)
    contrib = jax.lax.dot_general(hb, x16, (((0,), (0,)), ((), ())),
                                  preferred_element_type=jnp.float32)

    @pl.when(i == 0)
    def _init():
        eraw_ref[...] = contrib
        dege_ref[...] = de

    @pl.when(i > 0)
    def _acc():
        eraw_ref[...] += contrib
        dege_ref[...] += de


def _passB_body(h_ref, x32_ref, ew_ref, w_ref, bb_ref, e2_ref, *, v, tv, nv):
    i = pl.program_id(0)
    h = h_ref[...]
    if v % tv:
        h = jax.lax.cond(i == nv - 1,
                         lambda hh: _mask_rows(hh, i * tv, v),
                         lambda hh: hh, h)
    degv = jnp.clip(jnp.sum(h, axis=1, keepdims=True), 1.0, None)
    hb = h.astype(jnp.bfloat16)
    m = jax.lax.dot_general(hb, ew_ref[...], (((1,), (0,)), ((), ())),
                            preferred_element_type=jnp.float32) / degv
    r = jax.nn.relu(jnp.dot(m, w_ref[...], preferred_element_type=jnp.float32)
                    + bb_ref[...])
    xo = r + x32_ref[...]
    if v % tv:
        xo = jax.lax.cond(i == nv - 1,
                          lambda xx: _mask_rows(xx, i * tv, v),
                          lambda xx: xx, xo)
    contrib = jax.lax.dot_general(hb, xo.astype(jnp.bfloat16),
                                  (((0,), (0,)), ((), ())),
                                  preferred_element_type=jnp.float32)

    @pl.when(i == 0)
    def _init():
        e2_ref[...] = contrib

    @pl.when(i > 0)
    def _acc():
        e2_ref[...] += contrib


def _one_vocab(emb, g, b, W, bb, ew, H):
    v, d = emb.shape
    n_e = H.shape[1]
    tv = 256 if v >= 256 else 128
    nv = -(-v // tv)

    x32, x16 = pl.pallas_call(
        _bn_body,
        out_shape=[jax.ShapeDtypeStruct((v, d), jnp.float32),
                   jax.ShapeDtypeStruct((v, d), jnp.bfloat16)],
    )(emb, g, b)

    eraw, dege = pl.pallas_call(
        functools.partial(_passA_body, v=v, tv=tv, nv=nv),
        grid=(nv,),
        in_specs=[pl.BlockSpec((tv, n_e), lambda i: (i, 0)),
                  pl.BlockSpec((tv, d), lambda i: (i, 0))],
        out_specs=[pl.BlockSpec((n_e, d), lambda i: (0, 0)),
                   pl.BlockSpec((1, n_e), lambda i: (0, 0))],
        out_shape=[jax.ShapeDtypeStruct((n_e, d), jnp.float32),
                   jax.ShapeDtypeStruct((1, n_e), jnp.float32)],
    )(H, x16)

    inv_de = 1.0 / jnp.clip(dege[0], 1.0, None)
    ew_b = (eraw * (ew * inv_de)[:, None]).astype(jnp.bfloat16)

    e2raw = pl.pallas_call(
        functools.partial(_passB_body, v=v, tv=tv, nv=nv),
        grid=(nv,),
        in_specs=[pl.BlockSpec((tv, n_e), lambda i: (i, 0)),
                  pl.BlockSpec((tv, d), lambda i: (i, 0)),
                  pl.BlockSpec((n_e, d), lambda i: (0, 0)),
                  pl.BlockSpec((d, d), lambda i: (0, 0)),
                  pl.BlockSpec((1, d), lambda i: (0, 0))],
        out_specs=pl.BlockSpec((n_e, d), lambda i: (0, 0)),
        out_shape=jax.ShapeDtypeStruct((n_e, d), jnp.float32),
    )(H, x32, ew_b, W, bb[None, :])

    return e2raw * inv_de[:, None]


def kernel(emb_diag, g_diag, b_diag, W_diag, bb_diag, ew_diag,
           emb_proc, g_proc, b_proc, W_proc, bb_proc, ew_proc,
           emb_med, g_med, b_med, W_med, bb_med, ew_med,
           H_diag, H_proc, H_med):
    e2_diag = _one_vocab(emb_diag, g_diag, b_diag, W_diag, bb_diag, ew_diag, H_diag)
    e2_proc = _one_vocab(emb_proc, g_proc, b_proc, W_proc, bb_proc, ew_proc, H_proc)
    e2_med = _one_vocab(emb_med, g_med, b_med, W_med, bb_med, ew_med, H_med)
    return jnp.concatenate([e2_diag + e2_proc, e2_med], axis=-1)


# speedup vs baseline: 1.2819x; 1.2819x over previous
import functools

import jax
import jax.numpy as jnp
from jax.experimental import pallas as pl


def _bn_body(emb_ref, g_ref, b_ref, x32_ref, xt16_ref, *, v, v_pad):
    emb = emb_ref[...]
    mu = jnp.mean(emb, axis=0, keepdims=True)
    var = jnp.mean((emb - mu) ** 2, axis=0, keepdims=True)
    x = (emb - mu) * jax.lax.rsqrt(var + 1e-5) * g_ref[...] + b_ref[...]
    if v_pad > v:
        x32_ref[...] = jnp.concatenate(
            [x, jnp.zeros((v_pad - v, x.shape[1]), jnp.float32)], axis=0)
    else:
        x32_ref[...] = x
    xt16_ref[...] = jnp.swapaxes(x.astype(jnp.bfloat16), 0, 1)


def _passA_body(h_ref, xt_ref, et_ref, de_ref, h16_ref, degv_ref,
                *, v, v_pad, te, n_e, nte):
    i = pl.program_id(0)
    h = h_ref[...]
    rem = n_e % te
    if rem:
        def _mask(hh):
            lanes = jax.lax.broadcasted_iota(jnp.int32, hh.shape, 1)
            return jnp.where(lanes < rem, hh, 0.0)
        h = jax.lax.cond(i == nte - 1, _mask, lambda hh: hh, h)
    hb = h.astype(jnp.bfloat16)
    et = jax.lax.dot_general(xt_ref[...], hb, (((1,), (0,)), ((), ())),
                             preferred_element_type=jnp.float32)
    ones8 = jnp.ones((8, v), jnp.bfloat16)
    de = jax.lax.dot_general(ones8, hb, (((1,), (0,)), ((), ())),
                             preferred_element_type=jnp.float32)
    et_ref[...] = jnp.swapaxes(et, 0, 1)
    de_ref[...] = de
    dv = jnp.sum(h, axis=1, keepdims=True)
    if v_pad > v:
        hb = jnp.concatenate(
            [hb, jnp.zeros((v_pad - v, te), jnp.bfloat16)], axis=0)
        dv = jnp.concatenate(
            [dv, jnp.zeros((v_pad - v, 1), jnp.float32)], axis=0)
    h16_ref[...] = hb

    @pl.when(i == 0)
    def _init():
        degv_ref[...] = dv

    @pl.when(i > 0)
    def _acc():
        degv_ref[...] += dv


def _passB_body(h16_ref, x32_ref, ew_ref, w_ref, bb_ref, degv_ref, e2t_ref):
    i = pl.program_id(0)
    hb = h16_ref[...]
    m = jax.lax.dot_general(hb, ew_ref[...], (((1,), (0,)), ((), ())),
                            preferred_element_type=jnp.float32)
    m = m / jnp.clip(degv_ref[...], 1.0, None)
    r = jax.nn.relu(
        jax.lax.dot_general(m.astype(jnp.bfloat16), w_ref[...],
                            (((1,), (0,)), ((), ())),
                            preferred_element_type=jnp.float32) + bb_ref[...])
    xo16 = (r + x32_ref[...]).astype(jnp.bfloat16)
    xot = jnp.swapaxes(xo16, 0, 1)
    contrib = jax.lax.dot_general(xot, hb, (((1,), (0,)), ((), ())),
                                  preferred_element_type=jnp.float32)

    @pl.when(i == 0)
    def _init():
        e2t_ref[...] = contrib

    @pl.when(i > 0)
    def _acc():
        e2t_ref[...] += contrib


def _combine_body(ed_ref, ep_ref, em_ref, sd_ref, sp_ref, sm_ref, out_ref):
    dp = ed_ref[...] * sd_ref[0:1, :] + ep_ref[...] * sp_ref[0:1, :]
    mm = em_ref[...] * sm_ref[0:1, :]
    out_ref[...] = jnp.concatenate(
        [jnp.swapaxes(dp, 0, 1), jnp.swapaxes(mm, 0, 1)], axis=1)


def _one_vocab(emb, g, b, W, bb, ew, H):
    v, d = emb.shape
    n_e = H.shape[1]
    te = n_e if n_e <= 1024 else 1024
    nte = -(-n_e // te)
    tv = 512
    v_pad = -(-v // tv) * tv
    nv = v_pad // tv

    x32p, xt16 = pl.pallas_call(
        functools.partial(_bn_body, v=v, v_pad=v_pad),
        out_shape=[jax.ShapeDtypeStruct((v_pad, d), jnp.float32),
                   jax.ShapeDtypeStruct((d, v), jnp.bfloat16)],
    )(emb, g, b)

    et, de, h16, degv = pl.pallas_call(
        functools.partial(_passA_body, v=v, v_pad=v_pad, te=te, n_e=n_e,
                          nte=nte),
        grid=(nte,),
        in_specs=[pl.BlockSpec((v, te), lambda i: (0, i)),
                  pl.BlockSpec((d, v), lambda i: (0, 0))],
        out_specs=[pl.BlockSpec((te, d), lambda i: (i, 0)),
                   pl.BlockSpec((8, te), lambda i: (0, i)),
                   pl.BlockSpec((v_pad, te), lambda i: (0, i)),
                   pl.BlockSpec((v_pad, 1), lambda i: (0, 0))],
        out_shape=[jax.ShapeDtypeStruct((n_e, d), jnp.float32),
                   jax.ShapeDtypeStruct((8, n_e), jnp.float32),
                   jax.ShapeDtypeStruct((v_pad, n_e), jnp.bfloat16),
                   jax.ShapeDtypeStruct((v_pad, 1), jnp.float32)],
    )(H, xt16)

    de0 = jnp.clip(de[0], 1.0, None)
    ew16 = (et * (ew / de0)[:, None]).astype(jnp.bfloat16)

    e2t = pl.pallas_call(
        _passB_body,
        grid=(nv,),
        in_specs=[pl.BlockSpec((tv, n_e), lambda i: (i, 0)),
                  pl.BlockSpec((tv, d), lambda i: (i, 0)),
                  pl.BlockSpec((n_e, d), lambda i: (0, 0)),
                  pl.BlockSpec((d, d), lambda i: (0, 0)),
                  pl.BlockSpec((1, d), lambda i: (0, 0)),
                  pl.BlockSpec((tv, 1), lambda i: (i, 0))],
        out_specs=pl.BlockSpec((d, n_e), lambda i: (0, 0)),
        out_shape=jax.ShapeDtypeStruct((d, n_e), jnp.float32),
    )(h16, x32p, ew16, W.astype(jnp.bfloat16), bb[None, :], degv)

    return e2t, 1.0 / jnp.clip(de, 1.0, None)


def kernel(emb_diag, g_diag, b_diag, W_diag, bb_diag, ew_diag,
           emb_proc, g_proc, b_proc, W_proc, bb_proc, ew_proc,
           emb_med, g_med, b_med, W_med, bb_med, ew_med,
           H_diag, H_proc, H_med):
    e2t_d, s_d = _one_vocab(emb_diag, g_diag, b_diag, W_diag, bb_diag,
                            ew_diag, H_diag)
    e2t_p, s_p = _one_vocab(emb_proc, g_proc, b_proc, W_proc, bb_proc,
                            ew_proc, H_proc)
    e2t_m, s_m = _one_vocab(emb_med, g_med, b_med, W_med, bb_med,
                            ew_med, H_med)

    d, n_e = e2t_d.shape
    te = n_e if n_e <= 1024 else 1024
    nte = -(-n_e // te)
    return pl.pallas_call(
        _combine_body,
        grid=(nte,),
        in_specs=[pl.BlockSpec((d, te), lambda i: (0, i)),
                  pl.BlockSpec((d, te), lambda i: (0, i)),
                  pl.BlockSpec((d, te), lambda i: (0, i)),
                  pl.BlockSpec((8, te), lambda i: (0, i)),
                  pl.BlockSpec((8, te), lambda i: (0, i)),
                  pl.BlockSpec((8, te), lambda i: (0, i))],
        out_specs=pl.BlockSpec((te, 2 * d), lambda i: (i, 0)),
        out_shape=jax.ShapeDtypeStruct((n_e, 2 * d), jnp.float32),
    )(e2t_d, e2t_p, e2t_m, s_d, s_p, s_m)
